# Initial kernel scaffold; baseline (speedup 1.0000x reference)
import jax
import jax.numpy as jnp
from jax.experimental import pallas as pl


def kernel(position_ids, pe):
    raise NotImplementedError("write your pallas kernel here")


# speedup vs baseline: 4.3314x; 4.3314x over previous
import functools

import jax
import jax.numpy as jnp
from jax import lax
from jax.experimental import pallas as pl
from jax.experimental.pallas import tpu as pltpu
from jax.experimental.pallas import tpu_sc as plsc

_D = 128
_CHUNK = 128


@functools.lru_cache(maxsize=None)
def _build(total, n_workers, d):
    per_w = total // n_workers
    n_iter = per_w // _CHUNK
    mesh = plsc.VectorSubcoreMesh(core_axis_name="c", subcore_axis_name="s")

    @functools.partial(
        pl.kernel,
        mesh=mesh,
        out_type=jax.ShapeDtypeStruct((total, d), jnp.float32),
        scratch_types=[
            pltpu.VMEM((_CHUNK,), jnp.int32),
            pltpu.VMEM((_CHUNK, d), jnp.float32),
            pltpu.SemaphoreType.DMA,
        ],
    )
    def gather_kernel(ids_hbm, table_hbm, out_hbm, idx_v, rows_v, sem):
        wid = lax.axis_index("s") * 2 + lax.axis_index("c")
        base = wid * per_w

        def step(i, carry):
            off = base + i * _CHUNK
            pltpu.sync_copy(ids_hbm.at[pl.ds(off, _CHUNK)], idx_v)
            pltpu.async_copy(table_hbm.at[idx_v], rows_v, sem).wait()
            pltpu.sync_copy(rows_v, out_hbm.at[pl.ds(off, _CHUNK)])
            return carry

        lax.fori_loop(0, n_iter, step, 0)

    return gather_kernel


def kernel(position_ids, pe):
    b, t = position_ids.shape
    total = b * t
    flat_ids = position_ids.reshape(total)
    out = _build(total, 32, pe.shape[1])(flat_ids, pe)
    return out.reshape(b, t, pe.shape[1])


# speedup vs baseline: 4.6921x; 1.0833x over previous
import functools

import jax
import jax.numpy as jnp
from jax import lax
from jax.experimental import pallas as pl
from jax.experimental.pallas import tpu as pltpu
from jax.experimental.pallas import tpu_sc as plsc

_CHUNK = 128
_NBUF = 4


@functools.lru_cache(maxsize=None)
def _build(total, n_workers, d):
    per_w = total // n_workers
    n_iter = per_w // _CHUNK
    n_outer = n_iter // _NBUF
    assert n_iter == n_outer * _NBUF and n_outer >= 3
    mesh = plsc.VectorSubcoreMesh(core_axis_name="c", subcore_axis_name="s")

    @functools.partial(
        pl.kernel,
        mesh=mesh,
        out_type=jax.ShapeDtypeStruct((total, d), jnp.float32),
        scratch_types=[
            pltpu.VMEM((_NBUF, _CHUNK), jnp.int32),
            pltpu.VMEM((_NBUF, _CHUNK, d), jnp.float32),
        ]
        + [pltpu.SemaphoreType.DMA] * (3 * _NBUF),
    )
    def gather_kernel(ids_hbm, table_hbm, out_hbm, idx_v, rows_v, *sems):
        sem_i = sems[:_NBUF]
        sem_g = sems[_NBUF : 2 * _NBUF]
        sem_o = sems[2 * _NBUF :]
        wid = lax.axis_index("s") * 2 + lax.axis_index("c")
        base = wid * per_w

        def start_idx(b, g):
            pltpu.async_copy(
                ids_hbm.at[pl.ds(base + g * _CHUNK, _CHUNK)], idx_v.at[b], sem_i[b]
            )

        def wait_idx(b):
            pltpu.make_async_copy(
                ids_hbm.at[pl.ds(0, _CHUNK)], idx_v.at[b], sem_i[b]
            ).wait()

        def start_gather(b):
            pltpu.async_copy(table_hbm.at[idx_v.at[b]], rows_v.at[b], sem_g[b])

        def wait_gather(b):
            pltpu.make_async_copy(
                table_hbm.at[idx_v.at[b]], rows_v.at[b], sem_g[b]
            ).wait()

        def start_out(b, g):
            pltpu.async_copy(
                rows_v.at[b], out_hbm.at[pl.ds(base + g * _CHUNK, _CHUNK)], sem_o[b]
            )

        def wait_out(b):
            pltpu.make_async_copy(
                rows_v.at[b], out_hbm.at[pl.ds(0, _CHUNK)], sem_o[b]
            ).wait()

        for b in range(_NBUF):
            start_idx(b, b)

        for b in range(_NBUF):
            wait_idx(b)
            start_gather(b)
            if b > 0:
                wait_gather(b - 1)
                start_out(b - 1, b - 1)
                start_idx(b - 1, b - 1 + _NBUF)

        def outer(o, carry):
            for b in range(_NBUF):
                g = o * _NBUF + b
                wait_idx(b)
                wait_out(b)
                start_gather(b)
                bp = (b - 1) % _NBUF
                wait_gather(bp)
                start_out(bp, g - 1)
                start_idx(bp, g - 1 + _NBUF)
            return carry

        lax.fori_loop(1, n_outer - 1, outer, 0)

        for b in range(_NBUF):
            g = (n_outer - 1) * _NBUF + b
            wait_idx(b)
            wait_out(b)
            start_gather(b)
            bp = (b - 1) % _NBUF
            wait_gather(bp)
            start_out(bp, g - 1)
            if g - 1 + _NBUF < n_iter:
                start_idx(bp, g - 1 + _NBUF)

        wait_gather(_NBUF - 1)
        start_out(_NBUF - 1, n_iter - 1)
        for b in range(_NBUF):
            wait_out(b)

    return gather_kernel


def kernel(position_ids, pe):
    b, t = position_ids.shape
    total = b * t
    flat_ids = position_ids.reshape(total)
    out = _build(total, 32, pe.shape[1])(flat_ids, pe)
    return out.reshape(b, t, pe.shape[1])


# speedup vs baseline: 15.9654x; 3.4026x over previous
import functools

import jax
import jax.numpy as jnp
from jax import lax
from jax.experimental import pallas as pl
from jax.experimental.pallas import tpu as pltpu
from jax.experimental.pallas import tpu_sc as plsc

_CHUNK = 128
_NBUF = 4


@functools.lru_cache(maxsize=None)
def _build(total, n_workers, n_rows, d):
    per_w = total // n_workers
    n_iter = per_w // _CHUNK
    n_outer = n_iter // _NBUF
    assert n_iter == n_outer * _NBUF and n_outer >= 3
    mesh = plsc.VectorSubcoreMesh(core_axis_name="c", subcore_axis_name="s")

    @functools.partial(
        pl.kernel,
        mesh=mesh,
        out_type=jax.ShapeDtypeStruct((total, d), jnp.float32),
        scratch_types=[
            pltpu.VMEM((_NBUF, _CHUNK), jnp.int32),
            pltpu.VMEM((_NBUF, _CHUNK, d), jnp.float32),
            pltpu.VMEM_SHARED((n_rows, d), jnp.float32),
        ]
        + [pltpu.SemaphoreType.DMA] * (3 * _NBUF),
    )
    def gather_kernel(ids_hbm, table_hbm, out_hbm, idx_v, rows_v, table_sp, *sems):
        sem_i = sems[:_NBUF]
        sem_g = sems[_NBUF : 2 * _NBUF]
        sem_o = sems[2 * _NBUF :]
        sid = lax.axis_index("s")
        wid = sid * 2 + lax.axis_index("c")
        base = wid * per_w

        @pl.when(sid == 0)
        def _stage():
            pltpu.sync_copy(table_hbm, table_sp)

        plsc.subcore_barrier()

        def start_idx(b, g):
            pltpu.async_copy(
                ids_hbm.at[pl.ds(base + g * _CHUNK, _CHUNK)], idx_v.at[b], sem_i[b]
            )

        def wait_idx(b):
            pltpu.make_async_copy(
                ids_hbm.at[pl.ds(0, _CHUNK)], idx_v.at[b], sem_i[b]
            ).wait()

        def start_gather(b):
            pltpu.async_copy(table_sp.at[idx_v.at[b]], rows_v.at[b], sem_g[b])

        def wait_gather(b):
            pltpu.make_async_copy(
                table_sp.at[idx_v.at[b]], rows_v.at[b], sem_g[b]
            ).wait()

        def start_out(b, g):
            pltpu.async_copy(
                rows_v.at[b], out_hbm.at[pl.ds(base + g * _CHUNK, _CHUNK)], sem_o[b]
            )

        def wait_out(b):
            pltpu.make_async_copy(
                rows_v.at[b], out_hbm.at[pl.ds(0, _CHUNK)], sem_o[b]
            ).wait()

        for b in range(_NBUF):
            start_idx(b, b)

        for b in range(_NBUF):
            wait_idx(b)
            start_gather(b)
            if b > 0:
                wait_gather(b - 1)
                start_out(b - 1, b - 1)
                start_idx(b - 1, b - 1 + _NBUF)

        def outer(o, carry):
            for b in range(_NBUF):
                g = o * _NBUF + b
                wait_idx(b)
                wait_out(b)
                start_gather(b)
                bp = (b - 1) % _NBUF
                wait_gather(bp)
                start_out(bp, g - 1)
                start_idx(bp, g - 1 + _NBUF)
            return carry

        lax.fori_loop(1, n_outer - 1, outer, 0)

        for b in range(_NBUF):
            g = (n_outer - 1) * _NBUF + b
            wait_idx(b)
            wait_out(b)
            start_gather(b)
            bp = (b - 1) % _NBUF
            wait_gather(bp)
            start_out(bp, g - 1)
            if g - 1 + _NBUF < n_iter:
                start_idx(bp, g - 1 + _NBUF)

        wait_gather(_NBUF - 1)
        start_out(_NBUF - 1, n_iter - 1)
        for b in range(_NBUF):
            wait_out(b)

    return gather_kernel


def kernel(position_ids, pe):
    b, t = position_ids.shape
    total = b * t
    flat_ids = position_ids.reshape(total)
    out = _build(total, 32, pe.shape[0], pe.shape[1])(flat_ids, pe)
    return out.reshape(b, t, pe.shape[1])
